# baseline (speedup 1.0000x reference)
import functools

import jax
import jax.numpy as jnp
from jax.experimental import pallas as pl
from jax.experimental.pallas import tpu as pltpu

E, I, D, T, A = 8, 4096, 1024, 8, 2


def _ffn_kernel(idx_ref,
                x_ref,
                w1_ref, w3_ref,
                w2_ref,
                s1_ref, s3_ref,
                s2_ref,
                out_ref):
    e = pl.program_id(0)

    xb = x_ref[...].astype(jnp.bfloat16)
    dimn = (((1,), (1,)), ((), ()))
    w1 = w1_ref[0].astype(jnp.bfloat16)
    h1 = jax.lax.dot_general(xb, w1, dimn,
                             preferred_element_type=jnp.float32)
    w3 = w3_ref[0].astype(jnp.bfloat16)
    h3 = jax.lax.dot_general(xb, w3, dimn,
                             preferred_element_type=jnp.float32)
    g1 = h1 * s1_ref[0]
    x1 = g1 * jax.lax.logistic(g1)
    g = (x1 * (h3 * s3_ref[0])).astype(jnp.bfloat16)

    w2 = w2_ref[0].astype(jnp.bfloat16)
    y = jax.lax.dot_general(g, w2, dimn,
                            preferred_element_type=jnp.float32)

    yo = y * s2_ref[0]
    for t in range(T):
        for a in range(A):
            @pl.when(idx_ref[t, a] == e)
            def _():
                out_ref[t, a, :] = yo[t, :]


@jax.jit
def kernel(x, expert_indices, w1, w2, w3, scales1, scales2, scales3):
    idx = expert_indices.astype(jnp.int32)
    s1r = scales1.reshape(E, 1, I)
    s3r = scales3.reshape(E, 1, I)
    s2r = scales2.reshape(E, 1, D)
    out = pl.pallas_call(
        _ffn_kernel,
        grid=(E,),
        in_specs=[
            pl.BlockSpec(memory_space=pltpu.SMEM),
            pl.BlockSpec((T, D), lambda e: (0, 0)),
            pl.BlockSpec((1, I, D), lambda e: (e, 0, 0)),
            pl.BlockSpec((1, I, D), lambda e: (e, 0, 0)),
            pl.BlockSpec((1, D, I), lambda e: (e, 0, 0)),
            pl.BlockSpec((1, 1, I), lambda e: (e, 0, 0)),
            pl.BlockSpec((1, 1, I), lambda e: (e, 0, 0)),
            pl.BlockSpec((1, 1, D), lambda e: (e, 0, 0)),
        ],
        out_specs=pl.BlockSpec((T, A, D), lambda e: (0, 0, 0)),
        out_shape=jax.ShapeDtypeStruct((T, A, D), jnp.float32),
    )(idx, x, w1, w3, w2, s1r, s3r, s2r)
    return out


# speedup vs baseline: 1.3217x; 1.3217x over previous
import functools

import jax
import jax.numpy as jnp
from jax.experimental import pallas as pl
from jax.experimental.pallas import tpu as pltpu

E, I, D, T, A = 8, 4096, 1024, 8, 2


def _ffn_kernel(idx_ref,
                meta_ref,
                x_ref,
                w1_ref, w3_ref,
                w2_ref,
                s1_ref, s3_ref,
                s2_ref,
                out_ref):
    j = pl.program_id(0)
    e = meta_ref[j]
    cnt = meta_ref[8]

    @pl.when(j < cnt)
    def _():
        xb = x_ref[...].astype(jnp.bfloat16)
        dimn = (((1,), (1,)), ((), ()))
        w2 = w2_ref[0].astype(jnp.bfloat16)
        w1 = w1_ref[0].astype(jnp.bfloat16)
        h1 = jax.lax.dot_general(xb, w1, dimn,
                                 preferred_element_type=jnp.float32)
        w3 = w3_ref[0].astype(jnp.bfloat16)
        h3 = jax.lax.dot_general(xb, w3, dimn,
                                 preferred_element_type=jnp.float32)
        g1 = h1 * s1_ref[0]
        x1 = g1 * jax.lax.logistic(g1)
        g = (x1 * (h3 * s3_ref[0])).astype(jnp.bfloat16)

        y = jax.lax.dot_general(g, w2, dimn,
                                preferred_element_type=jnp.float32)

        yo = y * s2_ref[0]
        for t in range(T):
            for a in range(A):
                @pl.when(idx_ref[t * A + a] == e)
                def _():
                    out_ref[t, a, :] = yo[t, :]


@jax.jit
def kernel(x, expert_indices, w1, w2, w3, scales1, scales2, scales3):
    idx = expert_indices.astype(jnp.int32).reshape(-1)
    used = (idx[None, :] == jnp.arange(E, dtype=jnp.int32)[:, None]).any(axis=1)
    pos = jnp.cumsum(used) - 1
    cnt = pos[-1] + 1
    uniq = jnp.zeros(E, jnp.int32).at[jnp.where(used, pos, E)].set(
        jnp.arange(E, dtype=jnp.int32), mode="drop")
    uniq = uniq[jnp.minimum(jnp.arange(E), cnt - 1)]
    meta = jnp.concatenate([uniq, cnt.astype(jnp.int32)[None]])

    s1r = scales1.reshape(E, 1, I)
    s3r = scales3.reshape(E, 1, I)
    s2r = scales2.reshape(E, 1, D)

    grid_spec = pltpu.PrefetchScalarGridSpec(
        num_scalar_prefetch=2,
        grid=(E,),
        in_specs=[
            pl.BlockSpec((T, D), lambda j, idx_r, m_r: (0, 0)),
            pl.BlockSpec((1, I, D), lambda j, idx_r, m_r: (m_r[j], 0, 0)),
            pl.BlockSpec((1, I, D), lambda j, idx_r, m_r: (m_r[j], 0, 0)),
            pl.BlockSpec((1, D, I), lambda j, idx_r, m_r: (m_r[j], 0, 0)),
            pl.BlockSpec((1, 1, I), lambda j, idx_r, m_r: (m_r[j], 0, 0)),
            pl.BlockSpec((1, 1, I), lambda j, idx_r, m_r: (m_r[j], 0, 0)),
            pl.BlockSpec((1, 1, D), lambda j, idx_r, m_r: (m_r[j], 0, 0)),
        ],
        out_specs=pl.BlockSpec((T, A, D), lambda j, idx_r, m_r: (0, 0, 0)),
    )
    out = pl.pallas_call(
        _ffn_kernel,
        grid_spec=grid_spec,
        out_shape=jax.ShapeDtypeStruct((T, A, D), jnp.float32),
    )(idx, meta, x, w1, w3, w2, s1r, s3r, s2r)
    return out


# speedup vs baseline: 1.3662x; 1.0336x over previous
import functools

import jax
import jax.numpy as jnp
from jax.experimental import pallas as pl
from jax.experimental.pallas import tpu as pltpu

E, I, D, T, A = 8, 4096, 1024, 8, 2
P = T * A


def _route_kernel(idx_ref, meta_ref):
    count = jnp.int32(0)
    last = jnp.int32(0)
    for e in range(E):
        used = jnp.bool_(False)
        for p in range(P):
            used = used | (idx_ref[p] == e)

        @pl.when(used)
        def _():
            meta_ref[count] = jnp.int32(e)

        last = jnp.where(used, jnp.int32(e), last)
        count = count + used.astype(jnp.int32)
    for j in range(E):
        @pl.when(j >= count)
        def _():
            meta_ref[j] = last
    meta_ref[E] = count


def _ffn_kernel(idx_ref,
                meta_ref,
                x_ref,
                w1_ref, w3_ref,
                w2_ref,
                s1_ref, s3_ref,
                s2_ref,
                out_ref):
    j = pl.program_id(0)
    e = meta_ref[j]
    cnt = meta_ref[E]

    @pl.when(j < cnt)
    def _():
        xb = x_ref[...].astype(jnp.bfloat16)
        dimn = (((1,), (1,)), ((), ()))
        w2 = w2_ref[0].astype(jnp.bfloat16)
        w1 = w1_ref[0].astype(jnp.bfloat16)
        h1 = jax.lax.dot_general(xb, w1, dimn,
                                 preferred_element_type=jnp.float32)
        w3 = w3_ref[0].astype(jnp.bfloat16)
        h3 = jax.lax.dot_general(xb, w3, dimn,
                                 preferred_element_type=jnp.float32)
        g1 = h1 * s1_ref[0]
        x1 = g1 * jax.lax.logistic(g1)
        g = (x1 * (h3 * s3_ref[0])).astype(jnp.bfloat16)

        y = jax.lax.dot_general(g, w2, dimn,
                                preferred_element_type=jnp.float32)

        yo = y * s2_ref[0]
        for t in range(T):
            for a in range(A):
                @pl.when(idx_ref[t * A + a] == e)
                def _():
                    out_ref[t, a, :] = yo[t, :]


@jax.jit
def kernel(x, expert_indices, w1, w2, w3, scales1, scales2, scales3):
    idx = expert_indices.astype(jnp.int32).reshape(-1)
    meta = pl.pallas_call(
        _route_kernel,
        in_specs=[pl.BlockSpec(memory_space=pltpu.SMEM)],
        out_specs=pl.BlockSpec(memory_space=pltpu.SMEM),
        out_shape=jax.ShapeDtypeStruct((E + 1,), jnp.int32),
    )(idx)

    s1r = scales1.reshape(E, 1, I)
    s3r = scales3.reshape(E, 1, I)
    s2r = scales2.reshape(E, 1, D)

    grid_spec = pltpu.PrefetchScalarGridSpec(
        num_scalar_prefetch=2,
        grid=(E,),
        in_specs=[
            pl.BlockSpec((T, D), lambda j, idx_r, m_r: (0, 0)),
            pl.BlockSpec((1, I, D), lambda j, idx_r, m_r: (m_r[j], 0, 0)),
            pl.BlockSpec((1, I, D), lambda j, idx_r, m_r: (m_r[j], 0, 0)),
            pl.BlockSpec((1, D, I), lambda j, idx_r, m_r: (m_r[j], 0, 0)),
            pl.BlockSpec((1, 1, I), lambda j, idx_r, m_r: (m_r[j], 0, 0)),
            pl.BlockSpec((1, 1, I), lambda j, idx_r, m_r: (m_r[j], 0, 0)),
            pl.BlockSpec((1, 1, D), lambda j, idx_r, m_r: (m_r[j], 0, 0)),
        ],
        out_specs=pl.BlockSpec((T, A, D), lambda j, idx_r, m_r: (0, 0, 0)),
    )
    out = pl.pallas_call(
        _ffn_kernel,
        grid_spec=grid_spec,
        out_shape=jax.ShapeDtypeStruct((T, A, D), jnp.float32),
    )(idx, meta, x, w1, w3, w2, s1r, s3r, s2r)
    return out
